# baseline (speedup 1.0000x reference)
import functools

import jax
import jax.numpy as jnp
from jax import lax
from jax.experimental import pallas as pl
from jax.experimental.pallas import tpu as pltpu
from jax.experimental.pallas import tpu_sc as plsc

BATCH = 16384
EMBED = 32
HIDDEN = 64
LINE = 128
NROWS = 1000000

_info = plsc.get_sparse_core_info()
_NC, _NS = _info.num_cores, _info.num_subcores
_L = _info.num_lanes

_NT = (NROWS + LINE - 1) // LINE
_BASE_T = _NT // _NS
_REM_T = _NT % _NS
_CH = 1024
_MAXT = _BASE_T + 1
_NCHUNK = (_MAXT * LINE + _CH - 1) // _CH
_TAIL_START = (NROWS // LINE) * LINE
_TAIL = NROWS - _TAIL_START
_SB = 4

_mesh = plsc.VectorSubcoreMesh(core_axis_name="c", subcore_axis_name="s")

_i32 = jnp.int32


def _scan_table(ids_hbm, tbl_hbm, out_hbm, piece_v, lid_v, lj_v, b0, b1,
                tail_v, ring_v, dummy_v, sem_r, sem_c0, sem_c1, lo, hi):
    iota = lax.iota(_i32, _L)

    pltpu.sync_copy(ids_hbm.at[pl.ds(0, BATCH)], piece_v)

    def grp1(g, off):
        v = piece_v[pl.ds(g * _L, _L)]
        m = (v >= lo) & (v < hi)
        cnt = jnp.sum(m.astype(_i32))
        plsc.store_compressed(lid_v.at[pl.ds(off, _L)], v, mask=m)
        jvec = g * _L + iota
        plsc.store_compressed(lj_v.at[pl.ds(off, _L)], jvec, mask=m)
        return off + cnt

    n_loc = lax.fori_loop(0, BATCH // _L, grp1, jnp.asarray(0, _i32))

    def chunk_start(ci):
        return jnp.minimum(lo + ci * _CH, _TAIL_START - _CH)

    def process(buf, cs, cw, issued0):
        nb = (n_loc + _L - 1) // _L

        def grp(g, issued):
            base16 = g * _L
            v = lid_v[pl.ds(base16, _L)]
            okl = (base16 + iota) < n_loc
            m = (v >= cs) & (v < cs + cw) & okl
            cnt = jnp.sum(m.astype(_i32))

            def hits(hg):
                jg = lj_v[pl.ds(base16, _L)]
                jidx = jnp.where(m, jg, BATCH)
                lcl = jnp.minimum(jnp.maximum(v - cs, 0), cw - 1)
                p = hg & (_SB - 1)

                @pl.when(hg >= _SB)
                def _():
                    pltpu.make_async_copy(
                        out_hbm.at[pl.ds(0, _L)],
                        ring_v.at[pl.ds(0, _L)], sem_r).wait()

                for cc in range(EMBED):
                    ccv = jnp.broadcast_to(cc, (_L,))
                    col = plsc.load_gather(buf, [ccv, lcl])
                    plsc.store_scatter(ring_v, [p * _L + iota, ccv], col)
                pltpu.async_copy(ring_v.at[pl.ds(p * _L, _L)],
                                 out_hbm.at[jidx], sem_r)
                return hg + 1

            return lax.cond(cnt > 0, hits, lambda i: i, issued)

        return lax.fori_loop(0, nb, grp, issued0)

    pltpu.async_copy(tbl_hbm.at[:, pl.ds(chunk_start(0), _CH)], b0, sem_c0)
    pltpu.async_copy(tbl_hbm.at[:, pl.ds(chunk_start(1), _CH)], b1, sem_c1)

    def outer(ci0, issued):
        ci0 = ci0 * 2
        for b, (buf, sem) in enumerate(((b0, sem_c0), (b1, sem_c1))):
            ci = ci0 + b
            pltpu.make_async_copy(tbl_hbm.at[:, pl.ds(0, _CH)], buf,
                                  sem).wait()
            issued = process(buf, chunk_start(ci), _CH, issued)

            @pl.when(ci + 2 < _NCHUNK)
            def _():
                pltpu.async_copy(
                    tbl_hbm.at[:, pl.ds(chunk_start(ci + 2), _CH)], buf, sem)

        return issued

    issued = lax.fori_loop(0, _NCHUNK // 2, outer, jnp.asarray(0, _i32))
    if _NCHUNK % 2 == 1:
        pltpu.make_async_copy(tbl_hbm.at[:, pl.ds(0, _CH)], b0, sem_c0).wait()
        issued = process(b0, chunk_start(_NCHUNK - 1), _CH, issued)

    @pl.when(hi > NROWS)
    def _():
        pltpu.sync_copy(tbl_hbm.at[:, pl.ds(_TAIL_START, _TAIL)], tail_v)

    issued = lax.cond(
        hi > NROWS,
        lambda i: process(tail_v, jnp.asarray(_TAIL_START, _i32), _TAIL, i),
        lambda i: i, issued)

    def drain(_k, c2):
        pltpu.make_async_copy(out_hbm.at[pl.ds(0, _L)],
                              ring_v.at[pl.ds(0, _L)], sem_r).wait()
        return c2

    lax.fori_loop(0, jnp.minimum(issued, _SB), drain, 0)


@functools.partial(
    pl.kernel,
    mesh=_mesh,
    out_type=(
        jax.ShapeDtypeStruct((BATCH + 8, LINE), jnp.float32),
        jax.ShapeDtypeStruct((BATCH + 8, LINE), jnp.float32),
    ),
    scratch_types=[
        pltpu.VMEM((BATCH,), _i32),
        pltpu.VMEM((BATCH,), _i32),
        pltpu.VMEM((BATCH,), _i32),
        pltpu.VMEM((EMBED, _CH), jnp.float32),
        pltpu.VMEM((EMBED, _CH), jnp.float32),
        pltpu.VMEM((EMBED, _TAIL), jnp.float32),
        pltpu.VMEM((_SB * _L, LINE), jnp.float32),
        pltpu.VMEM((8, LINE), jnp.float32),
        pltpu.SemaphoreType.DMA,
        pltpu.SemaphoreType.DMA,
        pltpu.SemaphoreType.DMA,
    ],
    compiler_params=pltpu.CompilerParams(needs_layout_passes=False),
)
def _sc_scan(uid_hbm, pid_hbm, utT_hbm, ptT_hbm, uo_hbm, po_hbm,
             piece_v, lid_v, lj_v, b0, b1, tail_v, ring_v, dummy_v,
             sem_r, sem_c0, sem_c1):
    s = lax.axis_index("s")
    c = lax.axis_index("c")
    start_t = _BASE_T * s + jnp.minimum(s, _REM_T)
    ntiles = _BASE_T + jnp.minimum(jnp.maximum(_REM_T - s, 0), 1)
    lo = start_t * LINE
    hi = lo + ntiles * LINE

    @pl.when(c == 0)
    def _():
        _scan_table(uid_hbm, utT_hbm, uo_hbm, piece_v, lid_v, lj_v, b0, b1,
                    tail_v, ring_v, dummy_v, sem_r, sem_c0, sem_c1, lo, hi)

    @pl.when(c == 1)
    def _():
        _scan_table(pid_hbm, ptT_hbm, po_hbm, piece_v, lid_v, lj_v, b0, b1,
                    tail_v, ring_v, dummy_v, sem_r, sem_c0, sem_c1, lo, hi)


_BLK = 2048


def _mlp_body(ul_ref, pl_ref, w1u_ref, w1p_ref, b1_ref, w2_ref, b2_ref,
              o_ref):
    uemb = ul_ref[:, 0:EMBED]
    pemb = pl_ref[:, 0:EMBED]
    x = (jnp.dot(uemb, w1u_ref[...], preferred_element_type=jnp.float32)
         + jnp.dot(pemb, w1p_ref[...], preferred_element_type=jnp.float32)
         + b1_ref[...])
    x = jnp.maximum(x, 0.0)
    o_ref[...] = jnp.sum(x * w2_ref[...], axis=1, keepdims=True) + b2_ref[...]


def _mlp(u_lines, p_lines, w1u, w1p, b1_2d, w2t, b2_2d):
    grid = (BATCH // _BLK,)
    return pl.pallas_call(
        _mlp_body,
        grid=grid,
        in_specs=[
            pl.BlockSpec((_BLK, LINE), lambda i: (i, 0)),
            pl.BlockSpec((_BLK, LINE), lambda i: (i, 0)),
            pl.BlockSpec((EMBED, HIDDEN), lambda i: (0, 0)),
            pl.BlockSpec((EMBED, HIDDEN), lambda i: (0, 0)),
            pl.BlockSpec((1, HIDDEN), lambda i: (0, 0)),
            pl.BlockSpec((1, HIDDEN), lambda i: (0, 0)),
            pl.BlockSpec((1, 1), lambda i: (0, 0)),
        ],
        out_specs=pl.BlockSpec((_BLK, 1), lambda i: (i, 0)),
        out_shape=jax.ShapeDtypeStruct((BATCH, 1), jnp.float32),
    )(u_lines, p_lines, w1u, w1p, b1_2d, w2t, b2_2d)


def kernel(user_ids, post_ids, user_table, post_table, W1, b1, W2, b2):
    u_lines, p_lines = _sc_scan(user_ids, post_ids, user_table.T,
                                post_table.T)
    return _mlp(
        u_lines, p_lines,
        W1[:EMBED], W1[EMBED:],
        b1.reshape(1, HIDDEN),
        W2.reshape(1, HIDDEN),
        b2.reshape(1, 1),
    )


# speedup vs baseline: 13.6368x; 13.6368x over previous
import functools

import jax
import jax.numpy as jnp
from jax import lax
from jax.experimental import pallas as pl
from jax.experimental.pallas import tpu as pltpu
from jax.experimental.pallas import tpu_sc as plsc

BATCH = 16384
EMBED = 32
HIDDEN = 64
LINE = 128
RPL = LINE // EMBED
NROWS = 1000000
NLINES = NROWS // RPL

_info = plsc.get_sparse_core_info()
_NC, _NS = _info.num_cores, _info.num_subcores
_NW = _NC * _NS
_BPW = BATCH // _NW
_L = _info.num_lanes

_mesh = plsc.VectorSubcoreMesh(core_axis_name="c", subcore_axis_name="s")

_TCH = 4096

_NSB = (NROWS + _TCH - 1) // _TCH
_QPB = _TCH // RPL
_NLINES_P = _NSB * _QPB


def _pack_body(t_ref, eye_ref, o_ref):
    x = t_ref[...]
    e = eye_ref[...]
    parts = [
        lax.dot_general(
            x[:, a * _QPB:(a + 1) * _QPB], e,
            (((0,), (0,)), ((), ())),
            preferred_element_type=jnp.float32,
        )
        for a in range(RPL)
    ]
    o_ref[...] = jnp.concatenate(parts, axis=1)


def _pack(tT, eye):
    return pl.pallas_call(
        _pack_body,
        grid=(_NSB,),
        in_specs=[
            pl.BlockSpec((EMBED, _TCH), lambda i: (0, i)),
            pl.BlockSpec((EMBED, EMBED), lambda i: (0, 0)),
        ],
        out_specs=pl.BlockSpec((_QPB, LINE), lambda i: (i, 0)),
        out_shape=jax.ShapeDtypeStruct((_NLINES_P, LINE), jnp.float32),
    )(tT, eye)


@functools.partial(
    pl.kernel,
    mesh=_mesh,
    out_type=jax.ShapeDtypeStruct((BATCH, LINE), jnp.float32),
    scratch_types=[
        pltpu.VMEM((_BPW,), jnp.int32),
        pltpu.VMEM((_BPW,), jnp.int32),
        pltpu.VMEM((_BPW, LINE), jnp.float32),
        pltpu.SemaphoreType.DMA,
    ],
)
def _sc_gather(id_hbm, tbl_hbm, o_hbm, idx_v, lin_v, lines_v, sem):
    wid = lax.axis_index("s") * _NC + lax.axis_index("c")
    base = wid * _BPW
    pltpu.sync_copy(id_hbm.at[pl.ds(base, _BPW)], idx_v)

    def _shift(g, carry):
        s = pl.ds(g * _L, _L)
        v = idx_v[s]
        lin_v[s] = lax.shift_left(lax.shift_right_logical(v, 12), 10) | (v & (_QPB - 1))
        return carry

    lax.fori_loop(0, _BPW // _L, _shift, 0)

    pltpu.async_copy(tbl_hbm.at[lin_v], lines_v, sem).wait()
    pltpu.sync_copy(lines_v, o_hbm.at[pl.ds(base, _BPW)])


_BLK = 2048


def _mlp_body(ids_ref, ul_ref, pl_ref, w1u_ref, w1p_ref, b1_ref, w2_ref,
              b2_ref, o_ref):
    uoff = lax.shift_right_logical(ids_ref[:, 0:1], 10) & (RPL - 1)
    poff = lax.shift_right_logical(ids_ref[:, 1:2], 10) & (RPL - 1)
    lu = ul_ref[...]
    lp = pl_ref[...]
    uemb = (uoff == 0).astype(jnp.float32) * lu[:, 0:EMBED]
    pemb = (poff == 0).astype(jnp.float32) * lp[:, 0:EMBED]
    for k in range(1, RPL):
        uemb += (uoff == k).astype(jnp.float32) * lu[:, k * EMBED:(k + 1) * EMBED]
        pemb += (poff == k).astype(jnp.float32) * lp[:, k * EMBED:(k + 1) * EMBED]
    x = (jnp.dot(uemb, w1u_ref[...], preferred_element_type=jnp.float32)
         + jnp.dot(pemb, w1p_ref[...], preferred_element_type=jnp.float32)
         + b1_ref[...])
    x = jnp.maximum(x, 0.0)
    o_ref[...] = jnp.sum(x * w2_ref[...], axis=1, keepdims=True) + b2_ref[...]


def _mlp(ids2, u_lines, p_lines, w1u, w1p, b1_2d, w2t, b2_2d):
    grid = (BATCH // _BLK,)
    return pl.pallas_call(
        _mlp_body,
        grid=grid,
        in_specs=[
            pl.BlockSpec((_BLK, 2), lambda i: (i, 0)),
            pl.BlockSpec((_BLK, LINE), lambda i: (i, 0)),
            pl.BlockSpec((_BLK, LINE), lambda i: (i, 0)),
            pl.BlockSpec((EMBED, HIDDEN), lambda i: (0, 0)),
            pl.BlockSpec((EMBED, HIDDEN), lambda i: (0, 0)),
            pl.BlockSpec((1, HIDDEN), lambda i: (0, 0)),
            pl.BlockSpec((1, HIDDEN), lambda i: (0, 0)),
            pl.BlockSpec((1, 1), lambda i: (0, 0)),
        ],
        out_specs=pl.BlockSpec((_BLK, 1), lambda i: (i, 0)),
        out_shape=jax.ShapeDtypeStruct((BATCH, 1), jnp.float32),
    )(ids2, u_lines, p_lines, w1u, w1p, b1_2d, w2t, b2_2d)


def kernel(user_ids, post_ids, user_table, post_table, W1, b1, W2, b2):
    eye = jnp.eye(EMBED, dtype=jnp.float32)
    u_pack = _pack(user_table.T, eye)
    u_lines = _sc_gather(user_ids, u_pack)
    p_pack = _pack(post_table.T, eye)
    p_lines = _sc_gather(post_ids, p_pack)
    ids2 = jnp.stack([user_ids, post_ids], axis=1)
    return _mlp(
        ids2, u_lines, p_lines,
        W1[:EMBED], W1[EMBED:],
        b1.reshape(1, HIDDEN),
        W2.reshape(1, HIDDEN),
        b2.reshape(1, 1),
    )


# speedup vs baseline: 15.7563x; 1.1554x over previous
import functools

import jax
import jax.numpy as jnp
from jax import lax
from jax.experimental import pallas as pl
from jax.experimental.pallas import tpu as pltpu
from jax.experimental.pallas import tpu_sc as plsc

BATCH = 16384
EMBED = 32
HIDDEN = 64
LINE = 128
RPL = LINE // EMBED
NROWS = 1000000
NLINES = NROWS // RPL

_info = plsc.get_sparse_core_info()
_NC, _NS = _info.num_cores, _info.num_subcores
_NW = _NC * _NS
_BPW = BATCH // _NW
_L = _info.num_lanes

_mesh = plsc.VectorSubcoreMesh(core_axis_name="c", subcore_axis_name="s")

_TCH = 8192

_NSB = (NROWS + _TCH - 1) // _TCH
_QPB = _TCH // RPL
_NLINES_P = _NSB * _QPB


def _pack_body(t_ref, eye_ref, o_ref):
    x = t_ref[...]
    e = eye_ref[...]
    parts = [
        lax.dot_general(
            x[:, a * _QPB:(a + 1) * _QPB], e,
            (((0,), (0,)), ((), ())),
            preferred_element_type=jnp.float32,
        )
        for a in range(RPL)
    ]
    o_ref[...] = jnp.concatenate(parts, axis=1)


def _pack(tT, eye):
    return pl.pallas_call(
        _pack_body,
        grid=(_NSB,),
        in_specs=[
            pl.BlockSpec((EMBED, _TCH), lambda i: (0, i)),
            pl.BlockSpec((EMBED, EMBED), lambda i: (0, 0)),
        ],
        out_specs=pl.BlockSpec((_QPB, LINE), lambda i: (i, 0)),
        out_shape=jax.ShapeDtypeStruct((_NLINES_P, LINE), jnp.float32),
    )(tT, eye)


@functools.partial(
    pl.kernel,
    mesh=_mesh,
    out_type=jax.ShapeDtypeStruct((BATCH, LINE), jnp.float32),
    scratch_types=[
        pltpu.VMEM((_BPW,), jnp.int32),
        pltpu.VMEM((_BPW,), jnp.int32),
        pltpu.VMEM((_BPW, LINE), jnp.float32),
        pltpu.SemaphoreType.DMA,
    ],
)
def _sc_gather(id_hbm, tbl_hbm, o_hbm, idx_v, lin_v, lines_v, sem):
    wid = lax.axis_index("s") * _NC + lax.axis_index("c")
    base = wid * _BPW
    pltpu.sync_copy(id_hbm.at[pl.ds(base, _BPW)], idx_v)

    def _shift(g, carry):
        s = pl.ds(g * _L, _L)
        v = idx_v[s]
        lin_v[s] = lax.shift_left(lax.shift_right_logical(v, 12), 10) | (v & (_QPB - 1))
        return carry

    lax.fori_loop(0, _BPW // _L, _shift, 0)

    pltpu.async_copy(tbl_hbm.at[lin_v], lines_v, sem).wait()
    pltpu.sync_copy(lines_v, o_hbm.at[pl.ds(base, _BPW)])


_BLK = 1024


def _mlp_body(ids_ref, ul_ref, pl_ref, w1u_ref, w1p_ref, b1_ref, w2_ref,
              b2_ref, o_ref):
    uoff = lax.shift_right_logical(ids_ref[:, 0:1], 10) & (RPL - 1)
    poff = lax.shift_right_logical(ids_ref[:, 1:2], 10) & (RPL - 1)
    lu = ul_ref[...]
    lp = pl_ref[...]
    uemb = (uoff == 0).astype(jnp.float32) * lu[:, 0:EMBED]
    pemb = (poff == 0).astype(jnp.float32) * lp[:, 0:EMBED]
    for k in range(1, RPL):
        uemb += (uoff == k).astype(jnp.float32) * lu[:, k * EMBED:(k + 1) * EMBED]
        pemb += (poff == k).astype(jnp.float32) * lp[:, k * EMBED:(k + 1) * EMBED]
    x = (jnp.dot(uemb, w1u_ref[...], preferred_element_type=jnp.float32)
         + jnp.dot(pemb, w1p_ref[...], preferred_element_type=jnp.float32)
         + b1_ref[...])
    x = jnp.maximum(x, 0.0)
    o_ref[...] = jnp.sum(x * w2_ref[...], axis=1, keepdims=True) + b2_ref[...]


def _mlp(ids2, u_lines, p_lines, w1u, w1p, b1_2d, w2t, b2_2d):
    grid = (BATCH // _BLK,)
    return pl.pallas_call(
        _mlp_body,
        grid=grid,
        in_specs=[
            pl.BlockSpec((_BLK, 2), lambda i: (i, 0)),
            pl.BlockSpec((_BLK, LINE), lambda i: (i, 0)),
            pl.BlockSpec((_BLK, LINE), lambda i: (i, 0)),
            pl.BlockSpec((EMBED, HIDDEN), lambda i: (0, 0)),
            pl.BlockSpec((EMBED, HIDDEN), lambda i: (0, 0)),
            pl.BlockSpec((1, HIDDEN), lambda i: (0, 0)),
            pl.BlockSpec((1, HIDDEN), lambda i: (0, 0)),
            pl.BlockSpec((1, 1), lambda i: (0, 0)),
        ],
        out_specs=pl.BlockSpec((_BLK, 1), lambda i: (i, 0)),
        out_shape=jax.ShapeDtypeStruct((BATCH, 1), jnp.float32),
    )(ids2, u_lines, p_lines, w1u, w1p, b1_2d, w2t, b2_2d)


def kernel(user_ids, post_ids, user_table, post_table, W1, b1, W2, b2):
    eye = jnp.eye(EMBED, dtype=jnp.float32)
    u_pack = _pack(user_table.T, eye)
    u_lines = _sc_gather(user_ids, u_pack)
    p_pack = _pack(post_table.T, eye)
    p_lines = _sc_gather(post_ids, p_pack)
    ids2 = jnp.stack([user_ids, post_ids], axis=1)
    return _mlp(
        ids2, u_lines, p_lines,
        W1[:EMBED], W1[EMBED:],
        b1.reshape(1, HIDDEN),
        W2.reshape(1, HIDDEN),
        b2.reshape(1, 1),
    )


# speedup vs baseline: 15.7668x; 1.0007x over previous
import functools

import jax
import jax.numpy as jnp
from jax import lax
from jax.experimental import pallas as pl
from jax.experimental.pallas import tpu as pltpu
from jax.experimental.pallas import tpu_sc as plsc

BATCH = 16384
EMBED = 32
HIDDEN = 64
LINE = 128
RPL = LINE // EMBED
NROWS = 1000000
NLINES = NROWS // RPL

_info = plsc.get_sparse_core_info()
_NC, _NS = _info.num_cores, _info.num_subcores
_NW = _NC * _NS
_BPW = BATCH // _NW
_L = _info.num_lanes

_mesh = plsc.VectorSubcoreMesh(core_axis_name="c", subcore_axis_name="s")

_TCH = 8192

_NSB = (NROWS + _TCH - 1) // _TCH
_QPB = _TCH // RPL
_TCH_LOG = _TCH.bit_length() - 1
_QPB_LOG = _QPB.bit_length() - 1
_NLINES_P = _NSB * _QPB


def _pack_body(t_ref, eye_ref, o_ref):
    x = t_ref[...]
    e = eye_ref[...]
    parts = [
        lax.dot_general(
            x[:, a * _QPB:(a + 1) * _QPB], e,
            (((0,), (0,)), ((), ())),
            preferred_element_type=jnp.float32,
        )
        for a in range(RPL)
    ]
    o_ref[...] = jnp.concatenate(parts, axis=1)


def _pack(tT, eye):
    return pl.pallas_call(
        _pack_body,
        grid=(_NSB,),
        in_specs=[
            pl.BlockSpec((EMBED, _TCH), lambda i: (0, i)),
            pl.BlockSpec((EMBED, EMBED), lambda i: (0, 0)),
        ],
        out_specs=pl.BlockSpec((_QPB, LINE), lambda i: (i, 0)),
        out_shape=jax.ShapeDtypeStruct((_NLINES_P, LINE), jnp.float32),
    )(tT, eye)


@functools.partial(
    pl.kernel,
    mesh=_mesh,
    out_type=jax.ShapeDtypeStruct((BATCH, LINE), jnp.float32),
    scratch_types=[
        pltpu.VMEM((_BPW,), jnp.int32),
        pltpu.VMEM((_BPW,), jnp.int32),
        pltpu.VMEM((_BPW, LINE), jnp.float32),
        pltpu.SemaphoreType.DMA,
    ],
)
def _sc_gather(id_hbm, tbl_hbm, o_hbm, idx_v, lin_v, lines_v, sem):
    wid = lax.axis_index("s") * _NC + lax.axis_index("c")
    base = wid * _BPW
    pltpu.sync_copy(id_hbm.at[pl.ds(base, _BPW)], idx_v)

    def _shift(g, carry):
        s = pl.ds(g * _L, _L)
        v = idx_v[s]
        lin_v[s] = lax.shift_left(lax.shift_right_logical(v, _TCH_LOG), _QPB_LOG) | (v & (_QPB - 1))
        return carry

    lax.fori_loop(0, _BPW // _L, _shift, 0)

    pltpu.async_copy(tbl_hbm.at[lin_v], lines_v, sem).wait()
    pltpu.sync_copy(lines_v, o_hbm.at[pl.ds(base, _BPW)])


_BLK = 1024


def _mlp_body(ids_ref, ul_ref, pl_ref, w1u_ref, w1p_ref, b1_ref, w2_ref,
              b2_ref, o_ref):
    uoff = lax.shift_right_logical(ids_ref[:, 0:1], _QPB_LOG) & (RPL - 1)
    poff = lax.shift_right_logical(ids_ref[:, 1:2], _QPB_LOG) & (RPL - 1)
    lu = ul_ref[...]
    lp = pl_ref[...]
    uemb = (uoff == 0).astype(jnp.float32) * lu[:, 0:EMBED]
    pemb = (poff == 0).astype(jnp.float32) * lp[:, 0:EMBED]
    for k in range(1, RPL):
        uemb += (uoff == k).astype(jnp.float32) * lu[:, k * EMBED:(k + 1) * EMBED]
        pemb += (poff == k).astype(jnp.float32) * lp[:, k * EMBED:(k + 1) * EMBED]
    x = (jnp.dot(uemb, w1u_ref[...], preferred_element_type=jnp.float32)
         + jnp.dot(pemb, w1p_ref[...], preferred_element_type=jnp.float32)
         + b1_ref[...])
    x = jnp.maximum(x, 0.0)
    o_ref[...] = jnp.sum(x * w2_ref[...], axis=1, keepdims=True) + b2_ref[...]


def _mlp(ids2, u_lines, p_lines, w1u, w1p, b1_2d, w2t, b2_2d):
    grid = (BATCH // _BLK,)
    return pl.pallas_call(
        _mlp_body,
        grid=grid,
        in_specs=[
            pl.BlockSpec((_BLK, 2), lambda i: (i, 0)),
            pl.BlockSpec((_BLK, LINE), lambda i: (i, 0)),
            pl.BlockSpec((_BLK, LINE), lambda i: (i, 0)),
            pl.BlockSpec((EMBED, HIDDEN), lambda i: (0, 0)),
            pl.BlockSpec((EMBED, HIDDEN), lambda i: (0, 0)),
            pl.BlockSpec((1, HIDDEN), lambda i: (0, 0)),
            pl.BlockSpec((1, HIDDEN), lambda i: (0, 0)),
            pl.BlockSpec((1, 1), lambda i: (0, 0)),
        ],
        out_specs=pl.BlockSpec((_BLK, 1), lambda i: (i, 0)),
        out_shape=jax.ShapeDtypeStruct((BATCH, 1), jnp.float32),
    )(ids2, u_lines, p_lines, w1u, w1p, b1_2d, w2t, b2_2d)


def kernel(user_ids, post_ids, user_table, post_table, W1, b1, W2, b2):
    eye = jnp.eye(EMBED, dtype=jnp.float32)
    u_pack = _pack(user_table.T, eye)
    u_lines = _sc_gather(user_ids, u_pack)
    p_pack = _pack(post_table.T, eye)
    p_lines = _sc_gather(post_ids, p_pack)
    ids2 = jnp.stack([user_ids, post_ids], axis=1)
    return _mlp(
        ids2, u_lines, p_lines,
        W1[:EMBED], W1[EMBED:],
        b1.reshape(1, HIDDEN),
        W2.reshape(1, HIDDEN),
        b2.reshape(1, 1),
    )


# speedup vs baseline: 16.0194x; 1.0160x over previous
import functools

import jax
import jax.numpy as jnp
from jax import lax
from jax.experimental import pallas as pl
from jax.experimental.pallas import tpu as pltpu
from jax.experimental.pallas import tpu_sc as plsc

BATCH = 16384
EMBED = 32
HIDDEN = 64
LINE = 128
RPL = LINE // EMBED
NROWS = 1000000
NLINES = NROWS // RPL

_info = plsc.get_sparse_core_info()
_NC, _NS = _info.num_cores, _info.num_subcores
_NW = _NC * _NS
_BPW = BATCH // _NW
_L = _info.num_lanes

_mesh = plsc.VectorSubcoreMesh(core_axis_name="c", subcore_axis_name="s")

_TCH = 16384

_NSB = (NROWS + _TCH - 1) // _TCH
_QPB = _TCH // RPL
_TCH_LOG = _TCH.bit_length() - 1
_QPB_LOG = _QPB.bit_length() - 1
_NLINES_P = _NSB * _QPB


def _pack_body(t_ref, eye_ref, o_ref):
    x = t_ref[...]
    e = eye_ref[...]
    parts = [
        lax.dot_general(
            x[:, a * _QPB:(a + 1) * _QPB], e,
            (((0,), (0,)), ((), ())),
            preferred_element_type=jnp.float32,
        )
        for a in range(RPL)
    ]
    o_ref[...] = jnp.concatenate(parts, axis=1)


def _pack(tT, eye):
    return pl.pallas_call(
        _pack_body,
        grid=(_NSB,),
        in_specs=[
            pl.BlockSpec((EMBED, _TCH), lambda i: (0, i)),
            pl.BlockSpec((EMBED, EMBED), lambda i: (0, 0)),
        ],
        out_specs=pl.BlockSpec((_QPB, LINE), lambda i: (i, 0)),
        out_shape=jax.ShapeDtypeStruct((_NLINES_P, LINE), jnp.float32),
    )(tT, eye)


@functools.partial(
    pl.kernel,
    mesh=_mesh,
    out_type=jax.ShapeDtypeStruct((BATCH, LINE), jnp.float32),
    scratch_types=[
        pltpu.VMEM((_BPW,), jnp.int32),
        pltpu.VMEM((_BPW,), jnp.int32),
        pltpu.VMEM((_BPW, LINE), jnp.float32),
        pltpu.SemaphoreType.DMA,
    ],
)
def _sc_gather(id_hbm, tbl_hbm, o_hbm, idx_v, lin_v, lines_v, sem):
    wid = lax.axis_index("s") * _NC + lax.axis_index("c")
    base = wid * _BPW
    pltpu.sync_copy(id_hbm.at[pl.ds(base, _BPW)], idx_v)

    def _shift(g, carry):
        s = pl.ds(g * _L, _L)
        v = idx_v[s]
        lin_v[s] = lax.shift_left(lax.shift_right_logical(v, _TCH_LOG), _QPB_LOG) | (v & (_QPB - 1))
        return carry

    lax.fori_loop(0, _BPW // _L, _shift, 0)

    pltpu.async_copy(tbl_hbm.at[lin_v], lines_v, sem).wait()
    pltpu.sync_copy(lines_v, o_hbm.at[pl.ds(base, _BPW)])


_BLK = 1024


def _mlp_body(ids_ref, ul_ref, pl_ref, w1u_ref, w1p_ref, b1_ref, w2_ref,
              b2_ref, o_ref):
    uoff = lax.shift_right_logical(ids_ref[:, 0:1], _QPB_LOG) & (RPL - 1)
    poff = lax.shift_right_logical(ids_ref[:, 1:2], _QPB_LOG) & (RPL - 1)
    lu = ul_ref[...]
    lp = pl_ref[...]
    uemb = (uoff == 0).astype(jnp.float32) * lu[:, 0:EMBED]
    pemb = (poff == 0).astype(jnp.float32) * lp[:, 0:EMBED]
    for k in range(1, RPL):
        uemb += (uoff == k).astype(jnp.float32) * lu[:, k * EMBED:(k + 1) * EMBED]
        pemb += (poff == k).astype(jnp.float32) * lp[:, k * EMBED:(k + 1) * EMBED]
    x = (jnp.dot(uemb, w1u_ref[...], preferred_element_type=jnp.float32)
         + jnp.dot(pemb, w1p_ref[...], preferred_element_type=jnp.float32)
         + b1_ref[...])
    x = jnp.maximum(x, 0.0)
    o_ref[...] = jnp.sum(x * w2_ref[...], axis=1, keepdims=True) + b2_ref[...]


def _mlp(ids2, u_lines, p_lines, w1u, w1p, b1_2d, w2t, b2_2d):
    grid = (BATCH // _BLK,)
    return pl.pallas_call(
        _mlp_body,
        grid=grid,
        in_specs=[
            pl.BlockSpec((_BLK, 2), lambda i: (i, 0)),
            pl.BlockSpec((_BLK, LINE), lambda i: (i, 0)),
            pl.BlockSpec((_BLK, LINE), lambda i: (i, 0)),
            pl.BlockSpec((EMBED, HIDDEN), lambda i: (0, 0)),
            pl.BlockSpec((EMBED, HIDDEN), lambda i: (0, 0)),
            pl.BlockSpec((1, HIDDEN), lambda i: (0, 0)),
            pl.BlockSpec((1, HIDDEN), lambda i: (0, 0)),
            pl.BlockSpec((1, 1), lambda i: (0, 0)),
        ],
        out_specs=pl.BlockSpec((_BLK, 1), lambda i: (i, 0)),
        out_shape=jax.ShapeDtypeStruct((BATCH, 1), jnp.float32),
    )(ids2, u_lines, p_lines, w1u, w1p, b1_2d, w2t, b2_2d)


def kernel(user_ids, post_ids, user_table, post_table, W1, b1, W2, b2):
    eye = jnp.eye(EMBED, dtype=jnp.float32)
    u_pack = _pack(user_table.T, eye)
    u_lines = _sc_gather(user_ids, u_pack)
    p_pack = _pack(post_table.T, eye)
    p_lines = _sc_gather(post_ids, p_pack)
    ids2 = jnp.stack([user_ids, post_ids], axis=1)
    return _mlp(
        ids2, u_lines, p_lines,
        W1[:EMBED], W1[EMBED:],
        b1.reshape(1, HIDDEN),
        W2.reshape(1, HIDDEN),
        b2.reshape(1, 1),
    )


# speedup vs baseline: 16.1443x; 1.0078x over previous
import functools

import jax
import jax.numpy as jnp
from jax import lax
from jax.experimental import pallas as pl
from jax.experimental.pallas import tpu as pltpu
from jax.experimental.pallas import tpu_sc as plsc

BATCH = 16384
EMBED = 32
HIDDEN = 64
LINE = 128
RPL = LINE // EMBED
NROWS = 1000000
NLINES = NROWS // RPL

_info = plsc.get_sparse_core_info()
_NC, _NS = _info.num_cores, _info.num_subcores
_NW = _NC * _NS
_BPW = BATCH // _NW
_L = _info.num_lanes

_mesh = plsc.VectorSubcoreMesh(core_axis_name="c", subcore_axis_name="s")

_TCH = 32768

_NSB = (NROWS + _TCH - 1) // _TCH
_QPB = _TCH // RPL
_TCH_LOG = _TCH.bit_length() - 1
_QPB_LOG = _QPB.bit_length() - 1
_NLINES_P = _NSB * _QPB


def _pack_body(t_ref, eye_ref, o_ref):
    x = t_ref[...]
    e = eye_ref[...]
    parts = [
        lax.dot_general(
            x[:, a * _QPB:(a + 1) * _QPB], e,
            (((0,), (0,)), ((), ())),
            preferred_element_type=jnp.float32,
        )
        for a in range(RPL)
    ]
    o_ref[...] = jnp.concatenate(parts, axis=1)


def _pack(tT, eye):
    return pl.pallas_call(
        _pack_body,
        grid=(_NSB,),
        in_specs=[
            pl.BlockSpec((EMBED, _TCH), lambda i: (0, i)),
            pl.BlockSpec((EMBED, EMBED), lambda i: (0, 0)),
        ],
        out_specs=pl.BlockSpec((_QPB, LINE), lambda i: (i, 0)),
        out_shape=jax.ShapeDtypeStruct((_NLINES_P, LINE), jnp.float32),
    )(tT, eye)


@functools.partial(
    pl.kernel,
    mesh=_mesh,
    out_type=jax.ShapeDtypeStruct((BATCH, LINE), jnp.float32),
    scratch_types=[
        pltpu.VMEM((_BPW,), jnp.int32),
        pltpu.VMEM((_BPW,), jnp.int32),
        pltpu.VMEM((_BPW, LINE), jnp.float32),
        pltpu.SemaphoreType.DMA,
    ],
)
def _sc_gather(id_hbm, tbl_hbm, o_hbm, idx_v, lin_v, lines_v, sem):
    wid = lax.axis_index("s") * _NC + lax.axis_index("c")
    base = wid * _BPW
    pltpu.sync_copy(id_hbm.at[pl.ds(base, _BPW)], idx_v)

    def _shift(g, carry):
        s = pl.ds(g * _L, _L)
        v = idx_v[s]
        lin_v[s] = lax.shift_left(lax.shift_right_logical(v, _TCH_LOG), _QPB_LOG) | (v & (_QPB - 1))
        return carry

    lax.fori_loop(0, _BPW // _L, _shift, 0)

    pltpu.async_copy(tbl_hbm.at[lin_v], lines_v, sem).wait()
    pltpu.sync_copy(lines_v, o_hbm.at[pl.ds(base, _BPW)])


_BLK = 1024


def _mlp_body(ids_ref, ul_ref, pl_ref, w1u_ref, w1p_ref, b1_ref, w2_ref,
              b2_ref, o_ref):
    uoff = lax.shift_right_logical(ids_ref[:, 0:1], _QPB_LOG) & (RPL - 1)
    poff = lax.shift_right_logical(ids_ref[:, 1:2], _QPB_LOG) & (RPL - 1)
    lu = ul_ref[...]
    lp = pl_ref[...]
    uemb = (uoff == 0).astype(jnp.float32) * lu[:, 0:EMBED]
    pemb = (poff == 0).astype(jnp.float32) * lp[:, 0:EMBED]
    for k in range(1, RPL):
        uemb += (uoff == k).astype(jnp.float32) * lu[:, k * EMBED:(k + 1) * EMBED]
        pemb += (poff == k).astype(jnp.float32) * lp[:, k * EMBED:(k + 1) * EMBED]
    x = (jnp.dot(uemb, w1u_ref[...], preferred_element_type=jnp.float32)
         + jnp.dot(pemb, w1p_ref[...], preferred_element_type=jnp.float32)
         + b1_ref[...])
    x = jnp.maximum(x, 0.0)
    o_ref[...] = jnp.sum(x * w2_ref[...], axis=1, keepdims=True) + b2_ref[...]


def _mlp(ids2, u_lines, p_lines, w1u, w1p, b1_2d, w2t, b2_2d):
    grid = (BATCH // _BLK,)
    return pl.pallas_call(
        _mlp_body,
        grid=grid,
        in_specs=[
            pl.BlockSpec((_BLK, 2), lambda i: (i, 0)),
            pl.BlockSpec((_BLK, LINE), lambda i: (i, 0)),
            pl.BlockSpec((_BLK, LINE), lambda i: (i, 0)),
            pl.BlockSpec((EMBED, HIDDEN), lambda i: (0, 0)),
            pl.BlockSpec((EMBED, HIDDEN), lambda i: (0, 0)),
            pl.BlockSpec((1, HIDDEN), lambda i: (0, 0)),
            pl.BlockSpec((1, HIDDEN), lambda i: (0, 0)),
            pl.BlockSpec((1, 1), lambda i: (0, 0)),
        ],
        out_specs=pl.BlockSpec((_BLK, 1), lambda i: (i, 0)),
        out_shape=jax.ShapeDtypeStruct((BATCH, 1), jnp.float32),
    )(ids2, u_lines, p_lines, w1u, w1p, b1_2d, w2t, b2_2d)


def kernel(user_ids, post_ids, user_table, post_table, W1, b1, W2, b2):
    eye = jnp.eye(EMBED, dtype=jnp.float32)
    u_pack = _pack(user_table.T, eye)
    u_lines = _sc_gather(user_ids, u_pack)
    p_pack = _pack(post_table.T, eye)
    p_lines = _sc_gather(post_ids, p_pack)
    ids2 = jnp.stack([user_ids, post_ids], axis=1)
    return _mlp(
        ids2, u_lines, p_lines,
        W1[:EMBED], W1[EMBED:],
        b1.reshape(1, HIDDEN),
        W2.reshape(1, HIDDEN),
        b2.reshape(1, 1),
    )
